# baseline (speedup 1.0000x reference)
import math

import jax
import jax.numpy as jnp
from jax.experimental import pallas as pl
from jax.experimental.pallas import tpu as pltpu

_B, _S, _V = 64, 4, 100000
_LS = 0.1
_CONF = 1.0 - _LS
_SV = _LS / (_V - 2)
_ENT = (_V - 2) * _SV * math.log(_SV) + _CONF * math.log(_CONF)

_BB = 8
_NBLK = _B // _BB


def _loss_kernel(t_ref, x_ref, o_ref):
    i = pl.program_id(0)
    t = t_ref[...]
    x = x_ref[...]
    lane = jax.lax.broadcasted_iota(jnp.int32, (_BB, _S, _V), 2)
    sel = jnp.where(lane == t, -_CONF, -_SV)
    sel = jnp.where(lane == 0, 0.0, sel)
    rowvals = jnp.sum(x * sel, axis=2, keepdims=True)
    wrow = jnp.where(t == 0, 0.0, 1.0)
    partial = jnp.sum(wrow * (jnp.float32(_ENT) + rowvals))

    @pl.when(i == 0)
    def _():
        o_ref[0, 0] = partial

    @pl.when(i > 0)
    def _():
        o_ref[0, 0] = o_ref[0, 0] + partial


def kernel(output, target, one_hot):
    del one_hot
    t3 = target.reshape(_B, _S, 1)
    out = pl.pallas_call(
        _loss_kernel,
        grid=(_NBLK,),
        in_specs=[
            pl.BlockSpec((_BB, _S, 1), lambda i: (i, 0, 0)),
            pl.BlockSpec((_BB, _S, _V), lambda i: (i, 0, 0)),
        ],
        out_specs=pl.BlockSpec(memory_space=pltpu.SMEM),
        out_shape=jax.ShapeDtypeStruct((1, 1), jnp.float32),
        compiler_params=pltpu.CompilerParams(
            dimension_semantics=("arbitrary",),
        ),
    )(t3, output)
    return out[0, 0]


# speedup vs baseline: 1.0058x; 1.0058x over previous
import math

import jax
import jax.numpy as jnp
from jax.experimental import pallas as pl
from jax.experimental.pallas import tpu as pltpu

_B, _S, _V = 64, 4, 100000
_LS = 0.1
_CONF = 1.0 - _LS
_SV = _LS / (_V - 2)
_ENT = (_V - 2) * _SV * math.log(_SV) + _CONF * math.log(_CONF)

_BB = 4
_HALF = _B // 2
_NBLK = _HALF // _BB


def _chunk_partial(t, x):
    lane = jax.lax.broadcasted_iota(jnp.int32, (_BB, _S, _V), 2)
    sel = jnp.where(lane == t, -_CONF, -_SV)
    sel = jnp.where(lane == 0, 0.0, sel)
    rowvals = jnp.sum(x * sel, axis=2, keepdims=True)
    wrow = jnp.where(t == 0, 0.0, 1.0)
    return jnp.sum(wrow * (jnp.float32(_ENT) + rowvals))


def _loss_kernel(ta_ref, tb_ref, xa_ref, xb_ref, o_ref):
    i = pl.program_id(0)
    partial = (_chunk_partial(ta_ref[...], xa_ref[...])
               + _chunk_partial(tb_ref[...], xb_ref[...]))

    @pl.when(i == 0)
    def _():
        o_ref[0, 0] = partial

    @pl.when(i > 0)
    def _():
        o_ref[0, 0] = o_ref[0, 0] + partial


def kernel(output, target, one_hot):
    del one_hot
    t3 = target.reshape(_B, _S, 1)
    nh = _NBLK
    out = pl.pallas_call(
        _loss_kernel,
        grid=(_NBLK,),
        in_specs=[
            pl.BlockSpec((_BB, _S, 1), lambda i: (i, 0, 0)),
            pl.BlockSpec((_BB, _S, 1), lambda i: (i + nh, 0, 0)),
            pl.BlockSpec((_BB, _S, _V), lambda i: (i, 0, 0)),
            pl.BlockSpec((_BB, _S, _V), lambda i: (i + nh, 0, 0)),
        ],
        out_specs=pl.BlockSpec(memory_space=pltpu.SMEM),
        out_shape=jax.ShapeDtypeStruct((1, 1), jnp.float32),
        compiler_params=pltpu.CompilerParams(
            dimension_semantics=("arbitrary",),
        ),
    )(t3, t3, output, output)
    return out[0, 0]
